# Initial kernel scaffold; baseline (speedup 1.0000x reference)
import jax
import jax.numpy as jnp
from jax.experimental import pallas as pl


def kernel(x):
    raise NotImplementedError("write your pallas kernel here")


# speedup vs baseline: 1.5549x; 1.5549x over previous
import functools

import jax
import jax.numpy as jnp
import numpy as np
from jax.experimental import pallas as pl

_MASKING_RATE = 0.4
_MSK_SCALAR = -1.0


@functools.lru_cache(maxsize=None)
def _keep_mask(num_patches: int) -> np.ndarray:
    k = int(_MASKING_RATE * num_patches)
    with jax.ensure_compile_time_eval():
        mask_key = jax.random.fold_in(jax.random.key(0), 1)
        u = jax.random.uniform(mask_key, (num_patches,))
        idx = jnp.sort(jnp.argsort(u)[:k])
        masked = jnp.zeros((num_patches,), dtype=bool).at[idx].set(True)
        keep = np.asarray(~masked, dtype=np.float32).reshape(num_patches, 1)
    return keep


def _select_body(m_ref, x_ref, o_ref):
    m = m_ref[...][None, :, :]
    o_ref[...] = jnp.where(m != 0.0, x_ref[...], _MSK_SCALAR)


def kernel(x):
    B, P, D = x.shape
    keep = jnp.asarray(_keep_mask(P))
    PB = 1024
    grid = (B, P // PB)
    return pl.pallas_call(
        _select_body,
        grid=grid,
        in_specs=[
            pl.BlockSpec((PB, 1), lambda b, j: (j, 0)),
            pl.BlockSpec((1, PB, D), lambda b, j: (b, j, 0)),
        ],
        out_specs=pl.BlockSpec((1, PB, D), lambda b, j: (b, j, 0)),
        out_shape=jax.ShapeDtypeStruct((B, P, D), x.dtype),
    )(keep, x)
